# Initial kernel scaffold; baseline (speedup 1.0000x reference)
import jax
import jax.numpy as jnp
from jax.experimental import pallas as pl


def kernel(x, edge_index_ue, edge_index_ap, params):
    raise NotImplementedError("write your pallas kernel here")


# speedup vs baseline: 3.1146x; 3.1146x over previous
import functools
from math import sqrt

import jax
import jax.numpy as jnp
from jax import lax
from jax.experimental import pallas as pl
from jax.experimental.pallas import tpu as pltpu
from jax.experimental.pallas import tpu_sc as plsc

N_PAD = 10240
D = 128
E_PAD = 163840
EDGES_PER_W = E_PAD // 32
CHUNK = 64
CHUNKS = EDGES_PER_W // CHUNK
ROWS_PER_SUB = N_PAD // 16
INV_SQRT_D = 1.0 / sqrt(D)


def _mm8_body(x_ref, w_ref, b_ref, *out_refs):
    h = jnp.dot(x_ref[...], w_ref[...], preferred_element_type=jnp.float32)
    h = h + b_ref[0][None, :]
    for k in range(8):
        out_refs[k][...] = h[:, k * D:(k + 1) * D]


def _mm8(x, wt, b8):
    bm = N_PAD // 8
    return pl.pallas_call(
        _mm8_body,
        grid=(8,),
        in_specs=[
            pl.BlockSpec((bm, D), lambda i: (i, 0)),
            pl.BlockSpec((D, 8 * D), lambda i: (0, 0)),
            pl.BlockSpec((8, 8 * D), lambda i: (0, 0)),
        ],
        out_specs=[pl.BlockSpec((bm, D), lambda i: (i, 0)) for _ in range(8)],
        out_shape=[jax.ShapeDtypeStruct((N_PAD, D), jnp.float32) for _ in range(8)],
    )(x, wt, b8)


def _mm1_body(x_ref, w_ref, b_ref, out_ref):
    h = jnp.dot(x_ref[...], w_ref[...], preferred_element_type=jnp.float32)
    out_ref[...] = h + b_ref[0][None, :]


def _mm1(x, wt, b8):
    bm = N_PAD // 8
    return pl.pallas_call(
        _mm1_body,
        grid=(8,),
        in_specs=[
            pl.BlockSpec((bm, D), lambda i: (i, 0)),
            pl.BlockSpec((D, D), lambda i: (0, 0)),
            pl.BlockSpec((8, D), lambda i: (0, 0)),
        ],
        out_specs=pl.BlockSpec((bm, D), lambda i: (i, 0)),
        out_shape=jax.ShapeDtypeStruct((N_PAD, D), jnp.float32),
    )(x, wt, b8)


def _combine_body(oue_ref, due_ref, oap_ref, dap_ref, x1_ref, x5_ref,
                  g_ref, b_ref, out_ref):
    due = jnp.sum(due_ref[...], axis=0)
    dap = jnp.sum(dap_ref[...], axis=0)
    due = jnp.where(due == 0.0, 1.0, due)
    dap = jnp.where(dap == 0.0, 1.0, dap)
    oue = oue_ref[0] + oue_ref[1]
    oap = oap_ref[0] + oap_ref[1]
    o = (oue / due[:, None] + x1_ref[...] + oap / dap[:, None] + x5_ref[...])
    o = jnp.maximum(o, 0.0)
    m = jnp.mean(o, axis=1, keepdims=True)
    v = jnp.mean((o - m) * (o - m), axis=1, keepdims=True)
    out_ref[...] = (o - m) / jnp.sqrt(v + 1e-5) * g_ref[0][None, :] + b_ref[0][None, :]


def _combine(oue, due, oap, dap, x1, x5, g8, b8):
    bm = N_PAD // 8
    return pl.pallas_call(
        _combine_body,
        grid=(8,),
        in_specs=[
            pl.BlockSpec((2, bm, D), lambda i: (0, i, 0)),
            pl.BlockSpec((32, bm), lambda i: (0, i)),
            pl.BlockSpec((2, bm, D), lambda i: (0, i, 0)),
            pl.BlockSpec((32, bm), lambda i: (0, i)),
            pl.BlockSpec((bm, D), lambda i: (i, 0)),
            pl.BlockSpec((bm, D), lambda i: (i, 0)),
            pl.BlockSpec((8, D), lambda i: (0, 0)),
            pl.BlockSpec((8, D), lambda i: (0, 0)),
        ],
        out_specs=pl.BlockSpec((bm, D), lambda i: (i, 0)),
        out_shape=jax.ShapeDtypeStruct((N_PAD, D), jnp.float32),
    )(oue, due, oap, dap, x1, x5, g8, b8)


def _edge_body(a_hbm, b_hbm, c_hbm, ri_hbm, rj_hbm,
               out_hbm, den_hbm,
               idx_ri, idx_rj, arows, brows, crows, cscaled,
               den_part, acc, sem):
    c_ax = lax.axis_index("c")
    s_ax = lax.axis_index("s")
    wid = s_ax * 2 + c_ax
    rbase = s_ax * ROWS_PER_SUB

    z16 = jnp.zeros((16,), jnp.float32)

    def _zrow(i, carry):
        for t in range(8):
            cscaled[i, pl.ds(16 * t, 16)] = z16
        return carry
    lax.fori_loop(0, CHUNK, _zrow, 0)
    for q in range(ROWS_PER_SUB // CHUNK):
        pltpu.sync_copy(cscaled, acc.at[pl.ds(rbase + q * CHUNK, CHUNK)])

    def _zden(i, carry):
        den_part[pl.ds(i * 16, 16)] = z16
        return carry
    lax.fori_loop(0, N_PAD // 16, _zden, 0)

    plsc.subcore_barrier()

    ebase = wid * EDGES_PER_W
    lane = jnp.arange(16, dtype=jnp.int32)

    def _chunk(ch, carry):
        base = ebase + ch * CHUNK
        pltpu.sync_copy(ri_hbm.at[pl.ds(base, CHUNK)], idx_ri)
        pltpu.sync_copy(rj_hbm.at[pl.ds(base, CHUNK)], idx_rj)
        ha = pltpu.async_copy(a_hbm.at[idx_ri], arows, sem)
        hb = pltpu.async_copy(b_hbm.at[idx_rj], brows, sem)
        hc = pltpu.async_copy(c_hbm.at[idx_rj], crows, sem)
        ha.wait()
        hb.wait()
        hc.wait()

        for g in range(CHUNK // 16):
            def _edge(l, s16):
                e = g * 16 + l
                acc = arows[e, pl.ds(0, 16)] * brows[e, pl.ds(0, 16)]
                for t in range(1, 8):
                    acc = acc + arows[e, pl.ds(16 * t, 16)] * brows[e, pl.ds(16 * t, 16)]
                dot = jnp.sum(acc)
                sv = jnp.exp((jnp.zeros((16,), jnp.float32) + dot) * INV_SQRT_D)
                for t in range(8):
                    cscaled[e, pl.ds(16 * t, 16)] = crows[e, pl.ds(16 * t, 16)] * sv
                return jnp.where(lane == l, sv, s16)
            s16 = lax.fori_loop(0, 16, _edge, jnp.zeros((16,), jnp.float32))
            ri_g = idx_ri[pl.ds(16 * g, 16)]
            plsc.addupdate_scatter(den_part, [ri_g], s16)

        pltpu.sync_copy(cscaled, acc.at[idx_ri], add=True)
        return carry
    lax.fori_loop(0, CHUNKS, _chunk, 0)

    pltpu.sync_copy(den_part, den_hbm.at[wid])
    plsc.subcore_barrier()
    pltpu.sync_copy(acc.at[pl.ds(rbase, ROWS_PER_SUB)],
                    out_hbm.at[c_ax, pl.ds(rbase, ROWS_PER_SUB)])


def _edge_pass(a, b, c, ri, rj):
    mesh = plsc.VectorSubcoreMesh(core_axis_name="c", subcore_axis_name="s")
    fn = functools.partial(
        pl.kernel,
        mesh=mesh,
        compiler_params=pltpu.CompilerParams(needs_layout_passes=False),
        out_type=[
            jax.ShapeDtypeStruct((2, N_PAD, D), jnp.float32),
            jax.ShapeDtypeStruct((32, N_PAD), jnp.float32),
        ],
        scratch_types=[
            pltpu.VMEM((CHUNK,), jnp.int32),
            pltpu.VMEM((CHUNK,), jnp.int32),
            pltpu.VMEM((CHUNK, D), jnp.float32),
            pltpu.VMEM((CHUNK, D), jnp.float32),
            pltpu.VMEM((CHUNK, D), jnp.float32),
            pltpu.VMEM((CHUNK, D), jnp.float32),
            pltpu.VMEM((N_PAD,), jnp.float32),
            pltpu.VMEM_SHARED((N_PAD, D), jnp.float32),
            pltpu.SemaphoreType.DMA,
        ],
    )(_edge_body)
    return fn(a, b, c, ri, rj)


def kernel(x, edge_index_ue, edge_index_ap, params):
    n, d = x.shape
    xp = jnp.zeros((N_PAD, D), jnp.float32).at[:n].set(x)

    pad_n = E_PAD - edge_index_ue.shape[1]
    pad_ri = n + (jnp.arange(pad_n, dtype=jnp.int32) % (N_PAD - n))
    pad_rj = jnp.zeros((pad_n,), jnp.int32)

    def pad_edges(ei):
        return (jnp.concatenate([ei[0], pad_rj]),
                jnp.concatenate([ei[1], pad_ri]))

    rj_ue, ri_ue = pad_edges(edge_index_ue)
    rj_ap, ri_ap = pad_edges(edge_index_ap)

    for lp in params['layers']:
        wt = jnp.concatenate([lp['W' + str(k)] for k in range(1, 9)], axis=0).T
        b8 = jnp.broadcast_to(
            jnp.concatenate([lp['b' + str(k)] for k in range(1, 9)]), (8, 8 * D))
        x1, x2, x3, x4, x5, x6, x7, x8 = _mm8(xp, wt, b8)
        oue, due = _edge_pass(x3, x4, x2, ri_ue, rj_ue)
        oap, dap = _edge_pass(x7, x8, x6, ri_ap, rj_ap)
        g8 = jnp.broadcast_to(lp['gamma'], (8, D))
        be8 = jnp.broadcast_to(lp['beta'], (8, D))
        xp = _combine(oue, due, oap, dap, x1, x5, g8, be8)

    wlin_t = jnp.zeros((D, D), jnp.float32).at[:, :6].set(params['Wlin'].T)
    blin8 = jnp.broadcast_to(
        jnp.zeros((D,), jnp.float32).at[:6].set(params['blin']), (8, D))
    out = _mm1(xp, wlin_t, blin8)
    return out[:n, :6]


# speedup vs baseline: 5.5539x; 1.7832x over previous
import functools
from math import sqrt

import jax
import jax.numpy as jnp
from jax import lax
from jax.experimental import pallas as pl
from jax.experimental.pallas import tpu as pltpu
from jax.experimental.pallas import tpu_sc as plsc

N_PAD = 10240
D = 128
E_PAD = 163840
EDGES_PER_W = E_PAD // 32
CHUNK = 32
CHUNKS = EDGES_PER_W // CHUNK
ROWS_PER_SUB = N_PAD // 16
INV_SQRT_D = 1.0 / sqrt(D)


def _mm8_body(x_ref, w_ref, b_ref, *out_refs):
    h = jnp.dot(x_ref[...], w_ref[...], preferred_element_type=jnp.float32)
    h = h + b_ref[0][None, :]
    for k in range(8):
        out_refs[k][...] = h[:, k * D:(k + 1) * D]


def _mm8(x, wt, b8):
    bm = N_PAD // 8
    return pl.pallas_call(
        _mm8_body,
        grid=(8,),
        in_specs=[
            pl.BlockSpec((bm, D), lambda i: (i, 0)),
            pl.BlockSpec((D, 8 * D), lambda i: (0, 0)),
            pl.BlockSpec((8, 8 * D), lambda i: (0, 0)),
        ],
        out_specs=[pl.BlockSpec((bm, D), lambda i: (i, 0)) for _ in range(8)],
        out_shape=[jax.ShapeDtypeStruct((N_PAD, D), jnp.float32) for _ in range(8)],
    )(x, wt, b8)


def _mm1_body(x_ref, w_ref, b_ref, out_ref):
    h = jnp.dot(x_ref[...], w_ref[...], preferred_element_type=jnp.float32)
    out_ref[...] = h + b_ref[0][None, :]


def _mm1(x, wt, b8):
    bm = N_PAD // 8
    return pl.pallas_call(
        _mm1_body,
        grid=(8,),
        in_specs=[
            pl.BlockSpec((bm, D), lambda i: (i, 0)),
            pl.BlockSpec((D, D), lambda i: (0, 0)),
            pl.BlockSpec((8, D), lambda i: (0, 0)),
        ],
        out_specs=pl.BlockSpec((bm, D), lambda i: (i, 0)),
        out_shape=jax.ShapeDtypeStruct((N_PAD, D), jnp.float32),
    )(x, wt, b8)


def _combine_body(oue_ref, due_ref, oap_ref, dap_ref, x1_ref, x5_ref,
                  g_ref, b_ref, out_ref):
    due = due_ref[0] + due_ref[1]
    dap = dap_ref[0] + dap_ref[1]
    due = jnp.where(due == 0.0, 1.0, due)
    dap = jnp.where(dap == 0.0, 1.0, dap)
    oue = oue_ref[0] + oue_ref[1]
    oap = oap_ref[0] + oap_ref[1]
    o = (oue / due[:, None] + x1_ref[...] + oap / dap[:, None] + x5_ref[...])
    o = jnp.maximum(o, 0.0)
    m = jnp.mean(o, axis=1, keepdims=True)
    v = jnp.mean((o - m) * (o - m), axis=1, keepdims=True)
    out_ref[...] = (o - m) / jnp.sqrt(v + 1e-5) * g_ref[0][None, :] + b_ref[0][None, :]


def _combine(oue, due, oap, dap, x1, x5, g8, b8):
    bm = N_PAD // 8
    return pl.pallas_call(
        _combine_body,
        grid=(8,),
        in_specs=[
            pl.BlockSpec((2, bm, D), lambda i: (0, i, 0)),
            pl.BlockSpec((2, bm), lambda i: (0, i)),
            pl.BlockSpec((2, bm, D), lambda i: (0, i, 0)),
            pl.BlockSpec((2, bm), lambda i: (0, i)),
            pl.BlockSpec((bm, D), lambda i: (i, 0)),
            pl.BlockSpec((bm, D), lambda i: (i, 0)),
            pl.BlockSpec((8, D), lambda i: (0, 0)),
            pl.BlockSpec((8, D), lambda i: (0, 0)),
        ],
        out_specs=pl.BlockSpec((bm, D), lambda i: (i, 0)),
        out_shape=jax.ShapeDtypeStruct((N_PAD, D), jnp.float32),
    )(oue, due, oap, dap, x1, x5, g8, b8)


def _edge_body(a_hbm, b_hbm, c_hbm, idx_hbm,
               out_hbm, den_hbm,
               idxs, arows, brows, crows, cscaled, svec, zbuf,
               acc, den_acc,
               sem_g0, sem_g1, sem_s0, sem_s1):
    c_ax = lax.axis_index("c")
    s_ax = lax.axis_index("s")
    wid = s_ax * 2 + c_ax
    rbase = s_ax * ROWS_PER_SUB
    dbase = s_ax * (N_PAD // 16)
    sem_g = (sem_g0, sem_g1)
    sem_s = (sem_s0, sem_s1)

    z16 = jnp.zeros((16,), jnp.float32)
    lane = jnp.arange(16, dtype=jnp.int32)

    pltpu.sync_copy(idx_hbm.at[wid], idxs)

    def _zrow(i, carry):
        for t in range(8):
            cscaled[0][i, pl.ds(16 * t, 16)] = z16
        return carry
    lax.fori_loop(0, CHUNK, _zrow, 0)

    def _zden(i, carry):
        zbuf[pl.ds(i * 16, 16)] = z16
        return carry
    lax.fori_loop(0, (N_PAD // 16) // 16, _zden, 0)

    for q in range(ROWS_PER_SUB // CHUNK):
        pltpu.sync_copy(cscaled[0], acc.at[pl.ds(rbase + q * CHUNK, CHUNK)])
    pltpu.sync_copy(zbuf, den_acc.at[pl.ds(dbase, N_PAD // 16)])

    plsc.subcore_barrier()

    def _issue_gathers(g, b):
        pltpu.async_copy(a_hbm.at[idxs.at[2 * g]], arows[b], sem_g[b])
        pltpu.async_copy(b_hbm.at[idxs.at[2 * g + 1]], brows[b], sem_g[b])
        pltpu.async_copy(c_hbm.at[idxs.at[2 * g + 1]], crows[b], sem_g[b])

    def _wait_gathers(g, b):
        pltpu.make_async_copy(a_hbm.at[idxs.at[2 * g]], arows[b], sem_g[b]).wait()
        pltpu.make_async_copy(b_hbm.at[idxs.at[2 * g + 1]], brows[b], sem_g[b]).wait()
        pltpu.make_async_copy(c_hbm.at[idxs.at[2 * g + 1]], crows[b], sem_g[b]).wait()

    def _issue_scatters(g, b):
        pltpu.async_copy(cscaled[b], acc.at[idxs.at[2 * g]], sem_s[b], add=True)
        pltpu.async_copy(svec[b], den_acc.at[idxs.at[2 * g]], sem_s[b], add=True)

    def _wait_scatters(g, b):
        pltpu.make_async_copy(cscaled[b], acc.at[idxs.at[2 * g]], sem_s[b]).wait()
        pltpu.make_async_copy(svec[b], den_acc.at[idxs.at[2 * g]], sem_s[b]).wait()

    def _compute(b):
        for grp in range(CHUNK // 16):
            def _edge(l, s16):
                e = grp * 16 + l
                pacc = arows[b][e, pl.ds(0, 16)] * brows[b][e, pl.ds(0, 16)]
                for t in range(1, 8):
                    pacc = pacc + (arows[b][e, pl.ds(16 * t, 16)]
                                   * brows[b][e, pl.ds(16 * t, 16)])
                dot = jnp.sum(pacc)
                sv = jnp.exp((jnp.zeros((16,), jnp.float32) + dot) * INV_SQRT_D)
                for t in range(8):
                    cscaled[b][e, pl.ds(16 * t, 16)] = (
                        crows[b][e, pl.ds(16 * t, 16)] * sv)
                return jnp.where(lane == l, sv, s16)
            s16 = lax.fori_loop(0, 16, _edge, jnp.zeros((16,), jnp.float32))
            svec[b][pl.ds(16 * grp, 16)] = s16

    _issue_gathers(0, 0)

    def _pair(gp, carry):
        for b in range(2):
            g = 2 * gp + b
            _wait_gathers(g, b)
            if b == 0:
                _issue_gathers(g + 1, 1)
            else:
                @pl.when(gp < CHUNKS // 2 - 1)
                def _():
                    _issue_gathers(g + 1, 0)

            @pl.when(gp >= 1)
            def _():
                _wait_scatters(g - 2, b)
            _compute(b)
            _issue_scatters(g, b)
        return carry
    lax.fori_loop(0, CHUNKS // 2, _pair, 0)
    _wait_scatters(CHUNKS - 2, 0)
    _wait_scatters(CHUNKS - 1, 1)

    plsc.subcore_barrier()
    pltpu.sync_copy(acc.at[pl.ds(rbase, ROWS_PER_SUB)],
                    out_hbm.at[c_ax, pl.ds(rbase, ROWS_PER_SUB)])
    pltpu.sync_copy(den_acc.at[pl.ds(dbase, N_PAD // 16)],
                    den_hbm.at[c_ax, pl.ds(dbase, N_PAD // 16)])


def _edge_pass(a, b, c, idx_packed):
    mesh = plsc.VectorSubcoreMesh(core_axis_name="c", subcore_axis_name="s")
    fn = functools.partial(
        pl.kernel,
        mesh=mesh,
        compiler_params=pltpu.CompilerParams(
            needs_layout_passes=False, use_tc_tiling_on_sc=False),
        out_type=[
            jax.ShapeDtypeStruct((2, N_PAD, D), jnp.float32),
            jax.ShapeDtypeStruct((2, N_PAD), jnp.float32),
        ],
        scratch_types=[
            pltpu.VMEM((2 * CHUNKS, CHUNK), jnp.int32),
            [pltpu.VMEM((CHUNK, D), jnp.float32) for _ in range(2)],
            [pltpu.VMEM((CHUNK, D), jnp.float32) for _ in range(2)],
            [pltpu.VMEM((CHUNK, D), jnp.float32) for _ in range(2)],
            [pltpu.VMEM((CHUNK, D), jnp.float32) for _ in range(2)],
            [pltpu.VMEM((CHUNK,), jnp.float32) for _ in range(2)],
            pltpu.VMEM((N_PAD // 16,), jnp.float32),
            pltpu.VMEM_SHARED((N_PAD, D), jnp.float32),
            pltpu.VMEM_SHARED((N_PAD,), jnp.float32),
            pltpu.SemaphoreType.DMA,
            pltpu.SemaphoreType.DMA,
            pltpu.SemaphoreType.DMA,
            pltpu.SemaphoreType.DMA,
        ],
    )(_edge_body)
    return fn(a, b, c, idx_packed)


def kernel(x, edge_index_ue, edge_index_ap, params):
    n, d = x.shape
    xp = jnp.zeros((N_PAD, D), jnp.float32).at[:n].set(x)

    pad_n = E_PAD - edge_index_ue.shape[1]
    pad_ri = n + (jnp.arange(pad_n, dtype=jnp.int32) % (N_PAD - n))
    pad_rj = jnp.zeros((pad_n,), jnp.int32)

    def pack_edges(ei):
        rj = jnp.concatenate([ei[0], pad_rj]).reshape(32, CHUNKS, CHUNK)
        ri = jnp.concatenate([ei[1], pad_ri]).reshape(32, CHUNKS, CHUNK)
        return jnp.stack([ri, rj], axis=2).reshape(32, 2 * CHUNKS, CHUNK)

    idx_ue = pack_edges(edge_index_ue)
    idx_ap = pack_edges(edge_index_ap)

    for lp in params['layers']:
        wt = jnp.concatenate([lp['W' + str(k)] for k in range(1, 9)], axis=0).T
        b8 = jnp.broadcast_to(
            jnp.concatenate([lp['b' + str(k)] for k in range(1, 9)]), (8, 8 * D))
        x1, x2, x3, x4, x5, x6, x7, x8 = _mm8(xp, wt, b8)
        oue, due = _edge_pass(x3, x4, x2, idx_ue)
        oap, dap = _edge_pass(x7, x8, x6, idx_ap)
        g8 = jnp.broadcast_to(lp['gamma'], (8, D))
        be8 = jnp.broadcast_to(lp['beta'], (8, D))
        xp = _combine(oue, due, oap, dap, x1, x5, g8, be8)

    wlin_t = jnp.zeros((D, D), jnp.float32).at[:, :6].set(params['Wlin'].T)
    blin8 = jnp.broadcast_to(
        jnp.zeros((D,), jnp.float32).at[:6].set(params['blin']), (8, D))
    out = _mm1(xp, wlin_t, blin8)
    return out[:n, :6]
